# Initial kernel scaffold; baseline (speedup 1.0000x reference)
import jax
import jax.numpy as jnp
from jax import lax
from jax.experimental import pallas as pl
from jax.experimental.pallas import tpu as pltpu
from jax.experimental.pallas import tpu_sc as plsc

_N = 10000
_E = 320000
_D = 128
_NCLS = 40
_EPS = 1e-5

_NC, _NS = 2, 16
_NW = _NC * _NS
_EPW = _E // _NW
_CH = 80
_NCHUNK = _EPW // _CH
_DEGW = 16
_STRIPE = _N // _NS

_mesh = plsc.VectorSubcoreMesh(core_axis_name="c", subcore_axis_name="s")


def _make_agg(compute_deg: bool):
    out_type = [jax.ShapeDtypeStruct((_NC, _N, _D), jnp.float32)]
    scratch = [
        pltpu.VMEM((_CH,), jnp.int32),
        pltpu.VMEM((_CH,), jnp.int32),
        pltpu.VMEM((_CH, _D), jnp.float32),
        pltpu.VMEM_SHARED((_N, _D), jnp.float32),
        pltpu.SemaphoreType.DMA,
    ]
    if compute_deg:
        out_type.append(jax.ShapeDtypeStruct((_NC, _N, _DEGW), jnp.float32))
        scratch += [
            pltpu.VMEM((_CH, _DEGW), jnp.float32),
            pltpu.VMEM_SHARED((_N, _DEGW), jnp.float32),
        ]

    def body(h_hbm, src_hbm, dst_hbm, zeros_hbm, *refs):
        if compute_deg:
            (zdeg_hbm, ones_hbm, msg_out, deg_out,
             src_v, dst_v, rows_v, acc_sh, sem, ones_v, dega_sh) = refs
        else:
            (msg_out, src_v, dst_v, rows_v, acc_sh, sem) = refs
        c = lax.axis_index("c")
        s = lax.axis_index("s")
        wid = s * _NC + c
        row0 = s * _STRIPE

        pltpu.sync_copy(zeros_hbm.at[pl.ds(row0, _STRIPE)],
                        acc_sh.at[pl.ds(row0, _STRIPE)])
        if compute_deg:
            pltpu.sync_copy(zdeg_hbm.at[pl.ds(row0, _STRIPE)],
                            dega_sh.at[pl.ds(row0, _STRIPE)])
            pltpu.sync_copy(ones_hbm, ones_v)
        plsc.subcore_barrier()

        @pl.loop(0, _NCHUNK)
        def _(j):
            base = wid * _EPW + j * _CH
            pltpu.sync_copy(src_hbm.at[pl.ds(base, _CH)], src_v)
            pltpu.sync_copy(dst_hbm.at[pl.ds(base, _CH)], dst_v)
            pltpu.async_copy(h_hbm.at[src_v], rows_v, sem).wait()
            pltpu.sync_copy(rows_v, acc_sh.at[dst_v], add=True)
            if compute_deg:
                pltpu.sync_copy(ones_v, dega_sh.at[dst_v], add=True)

        plsc.subcore_barrier()
        pltpu.sync_copy(acc_sh.at[pl.ds(row0, _STRIPE)],
                        msg_out.at[c, pl.ds(row0, _STRIPE)])
        if compute_deg:
            pltpu.sync_copy(dega_sh.at[pl.ds(row0, _STRIPE)],
                            deg_out.at[c, pl.ds(row0, _STRIPE)])

    return pl.kernel(body, out_type=out_type, mesh=_mesh, scratch_types=scratch)


_agg_deg = _make_agg(True)
_agg = _make_agg(False)


def _tc_layer_body(h_ref, p_ref, degp_ref, wt_ref, wb_ref, b_ref, g_ref, be_ref,
                   o_ref):
    deg = degp_ref[0, :, 0:1] + degp_ref[1, :, 0:1]
    recip = 1.0 / jnp.maximum(deg, 1.0)
    hn = (p_ref[0] + p_ref[1]) * recip
    h = h_ref[...]
    z = (jnp.dot(h, wt_ref[...], preferred_element_type=jnp.float32)
         + jnp.dot(hn, wb_ref[...], preferred_element_type=jnp.float32)
         + b_ref[...])
    r = jnp.maximum(z, 0.0)
    m = jnp.mean(r, axis=0, keepdims=True)
    v = jnp.mean((r - m) * (r - m), axis=0, keepdims=True)
    o_ref[...] = h + (r - m) * lax.rsqrt(v + _EPS) * g_ref[...] + be_ref[...]


def _tc_out_body(h_ref, p_ref, degp_ref, wt_ref, wb_ref, b_ref, o_ref):
    deg = degp_ref[0, :, 0:1] + degp_ref[1, :, 0:1]
    recip = 1.0 / jnp.maximum(deg, 1.0)
    hn = (p_ref[0] + p_ref[1]) * recip
    o_ref[...] = (jnp.dot(h_ref[...], wt_ref[...],
                          preferred_element_type=jnp.float32)
                  + jnp.dot(hn, wb_ref[...], preferred_element_type=jnp.float32)
                  + b_ref[...])


_tc_layer = pl.pallas_call(
    _tc_layer_body,
    out_shape=jax.ShapeDtypeStruct((_N, _D), jnp.float32),
)

_tc_out = pl.pallas_call(
    _tc_out_body,
    out_shape=jax.ShapeDtypeStruct((_N, _NCLS), jnp.float32),
)


@jax.jit
def kernel(h, edge_index, e, W0, b0, gamma0, beta0, W1, b1, gamma1, beta1,
           W2, b2):
    del e
    src = edge_index[0]
    dst = edge_index[1]
    zeros_nd = jnp.zeros((_N, _D), jnp.float32)
    zeros_ndeg = jnp.zeros((_N, _DEGW), jnp.float32)
    ones_ch = jnp.ones((_CH, _DEGW), jnp.float32)

    parts0, degp = _agg_deg(h, src, dst, zeros_nd, zeros_ndeg, ones_ch)
    h0 = _tc_layer(h, parts0, degp, W0[:_D], W0[_D:], b0, gamma0, beta0)
    parts1 = _agg(h0, src, dst, zeros_nd)
    h1 = _tc_layer(h0, parts1, degp, W1[:_D], W1[_D:], b1, gamma1, beta1)
    parts2 = _agg(h1, src, dst, zeros_nd)
    return _tc_out(h1, parts2, degp, W2[:_D], W2[_D:], b2)


# speedup vs baseline: 4.3240x; 4.3240x over previous
import functools

import jax
import jax.numpy as jnp
from jax import lax
from jax.experimental import pallas as pl
from jax.experimental.pallas import tpu as pltpu
from jax.experimental.pallas import tpu_sc as plsc

_N = 10000
_E = 320000
_D = 128
_NCLS = 40
_EPS = 1e-5

_NC, _NS = 2, 16
_NW = _NC * _NS
_EPW = _E // _NW
_CH = 80
_NCHUNK = _EPW // _CH
_STRIPE = 624
_TAIL0 = _NS * _STRIPE
_TAIL = _N - _TAIL0


def _stripe_copy(src, dst, row0, s):
    pltpu.sync_copy(src.at[pl.ds(row0, _STRIPE)], dst.at[pl.ds(row0, _STRIPE)])

    @pl.when(s == 0)
    def _():
        pltpu.sync_copy(src.at[pl.ds(_TAIL0, _TAIL)],
                        dst.at[pl.ds(_TAIL0, _TAIL)])


@functools.lru_cache(maxsize=None)
def _make_agg(with_gather: bool):
    scratch = [
        pltpu.VMEM((_CH,), jnp.int32),
        pltpu.VMEM((_CH, _D), jnp.float32),
        pltpu.VMEM_SHARED((_N, _D), jnp.float32),
        pltpu.SemaphoreType.DMA,
    ]
    if with_gather:
        scratch.insert(0, pltpu.VMEM((_CH,), jnp.int32))

    def body(*refs):
        if with_gather:
            (h_hbm, src_hbm, dst_hbm, zeros_hbm, msg_out,
             src_v, dst_v, rows_v, acc_sh, sem) = refs
        else:
            (ones_hbm, dst_hbm, zeros_hbm, msg_out,
             dst_v, rows_v, acc_sh, sem) = refs
        c = lax.axis_index("c")
        s = lax.axis_index("s")
        wid = s * _NC + c
        row0 = s * _STRIPE

        _stripe_copy(zeros_hbm, acc_sh, row0, s)
        if not with_gather:
            pltpu.sync_copy(ones_hbm, rows_v)
        plsc.subcore_barrier()

        @pl.loop(0, _NCHUNK)
        def _(j):
            base = wid * _EPW + j * _CH
            pltpu.sync_copy(dst_hbm.at[pl.ds(base, _CH)], dst_v)
            if with_gather:
                pltpu.sync_copy(src_hbm.at[pl.ds(base, _CH)], src_v)
                pltpu.async_copy(h_hbm.at[src_v], rows_v, sem).wait()
            pltpu.sync_copy(rows_v, acc_sh.at[dst_v], add=True)

        plsc.subcore_barrier()
        _stripe_copy(acc_sh, msg_out.at[c], row0, s)

    mesh = plsc.VectorSubcoreMesh(core_axis_name="c", subcore_axis_name="s")
    return pl.kernel(body,
                     out_type=jax.ShapeDtypeStruct((_NC, _N, _D), jnp.float32),
                     mesh=mesh, scratch_types=scratch)


def _tc_layer_body(h_ref, p_ref, degp_ref, wt_ref, wb_ref, b_ref, g_ref, be_ref,
                   o_ref):
    deg = degp_ref[0, :, 0:1] + degp_ref[1, :, 0:1]
    recip = 1.0 / jnp.maximum(deg, 1.0)
    hn = (p_ref[0] + p_ref[1]) * recip
    h = h_ref[...]
    z = (jnp.dot(h, wt_ref[...], preferred_element_type=jnp.float32)
         + jnp.dot(hn, wb_ref[...], preferred_element_type=jnp.float32)
         + b_ref[...])
    r = jnp.maximum(z, 0.0)
    m = jnp.mean(r, axis=0, keepdims=True)
    v = jnp.mean((r - m) * (r - m), axis=0, keepdims=True)
    o_ref[...] = h + (r - m) * lax.rsqrt(v + _EPS) * g_ref[...] + be_ref[...]


def _tc_out_body(h_ref, p_ref, degp_ref, wt_ref, wb_ref, b_ref, o_ref):
    deg = degp_ref[0, :, 0:1] + degp_ref[1, :, 0:1]
    recip = 1.0 / jnp.maximum(deg, 1.0)
    hn = (p_ref[0] + p_ref[1]) * recip
    o_ref[...] = (jnp.dot(h_ref[...], wt_ref[...],
                          preferred_element_type=jnp.float32)
                  + jnp.dot(hn, wb_ref[...], preferred_element_type=jnp.float32)
                  + b_ref[...])


_tc_layer = pl.pallas_call(
    _tc_layer_body,
    out_shape=jax.ShapeDtypeStruct((_N, _D), jnp.float32),
)

_tc_out = pl.pallas_call(
    _tc_out_body,
    out_shape=jax.ShapeDtypeStruct((_N, _NCLS), jnp.float32),
)


@jax.jit
def kernel(h, edge_index, e, W0, b0, gamma0, beta0, W1, b1, gamma1, beta1,
           W2, b2):
    del e
    src = edge_index[0]
    dst = edge_index[1]
    zeros_nd = jnp.zeros((_N, _D), jnp.float32)
    ones_ch = jnp.ones((_CH, _D), jnp.float32)

    degp = _make_agg(False)(ones_ch, dst, zeros_nd)
    parts0 = _make_agg(True)(h, src, dst, zeros_nd)
    h0 = _tc_layer(h, parts0, degp, W0[:_D], W0[_D:], b0, gamma0, beta0)
    parts1 = _make_agg(True)(h0, src, dst, zeros_nd)
    h1 = _tc_layer(h0, parts1, degp, W1[:_D], W1[_D:], b1, gamma1, beta1)
    parts2 = _make_agg(True)(h1, src, dst, zeros_nd)
    return _tc_out(h1, parts2, degp, W2[:_D], W2[_D:], b2)


# speedup vs baseline: 7.2430x; 1.6751x over previous
import functools

import jax
import jax.numpy as jnp
from jax import lax
from jax.experimental import pallas as pl
from jax.experimental.pallas import tpu as pltpu
from jax.experimental.pallas import tpu_sc as plsc

_N = 10000
_E = 320000
_D = 128
_NCLS = 40
_EPS = 1e-5

_NC, _NS = 2, 16
_NW = _NC * _NS
_EPW = _E // _NW
_CH = 80
_NCHUNK = _EPW // _CH
_STRIPE = 624
_TAIL0 = _NS * _STRIPE
_TAIL = _N - _TAIL0


def _stripe_copy(src, dst, row0, s):
    pltpu.sync_copy(src.at[pl.ds(row0, _STRIPE)], dst.at[pl.ds(row0, _STRIPE)])

    @pl.when(s == 0)
    def _():
        pltpu.sync_copy(src.at[pl.ds(_TAIL0, _TAIL)],
                        dst.at[pl.ds(_TAIL0, _TAIL)])


_MCH = 80
_MNCHUNK = _EPW // _MCH


@functools.lru_cache(maxsize=None)
def _make_msg():
    scratch = [
        pltpu.VMEM((_MCH,), jnp.int32),
        pltpu.VMEM((_MCH,), jnp.int32),
        pltpu.VMEM((_MCH,), jnp.int32),
        pltpu.VMEM((_MCH,), jnp.int32),
        pltpu.VMEM((_MCH, _D), jnp.float32),
        pltpu.VMEM((_MCH, _D), jnp.float32),
        pltpu.VMEM_SHARED((_N, _D), jnp.float32),
        pltpu.SemaphoreType.DMA,
        pltpu.SemaphoreType.DMA,
        pltpu.SemaphoreType.DMA,
        pltpu.SemaphoreType.DMA,
        pltpu.SemaphoreType.DMA,
        pltpu.SemaphoreType.DMA,
        pltpu.SemaphoreType.DMA,
        pltpu.SemaphoreType.DMA,
    ]

    def body(h_hbm, src_hbm, dst_hbm, zeros_hbm, msg_out,
             src0, src1, dst0, dst1, rows0, rows1, acc_sh,
             sg0, sg1, ss0, ss1, si0, si1, sd0, sd1):
        c = lax.axis_index("c")
        s = lax.axis_index("s")
        wid = s * _NC + c
        row0 = s * _STRIPE
        base = wid * _EPW
        src = (src0, src1)
        dst = (dst0, dst1)
        rows = (rows0, rows1)
        sg = (sg0, sg1)
        ss = (ss0, ss1)
        si = (si0, si1)
        sd = (sd0, sd1)

        _stripe_copy(zeros_hbm, acc_sh, row0, s)
        plsc.subcore_barrier()

        def load_src(k, p):
            pltpu.async_copy(src_hbm.at[pl.ds(base + k * _MCH, _MCH)],
                             src[p], si[p])

        def load_dst(k, p):
            pltpu.async_copy(dst_hbm.at[pl.ds(base + k * _MCH, _MCH)],
                             dst[p], sd[p])

        def wait_idx(sem, buf):
            pltpu.make_async_copy(src_hbm.at[pl.ds(0, _MCH)], buf, sem).wait()

        def gather(p):
            pltpu.async_copy(h_hbm.at[src[p]], rows[p], sg[p])

        def wait_g(p):
            pltpu.make_async_copy(h_hbm.at[src[p]], rows[p], sg[p]).wait()

        def scatter(p):
            pltpu.async_copy(rows[p], acc_sh.at[dst[p]], ss[p], add=True)

        def wait_s(p):
            pltpu.make_async_copy(rows[p], acc_sh.at[dst[p]], ss[p]).wait()

        def step(k, p, first=False, do_src=True):
            q = 1 - p
            wait_g(p)
            wait_idx(sd[p], dst[p])
            scatter(p)
            if do_src:
                load_src(k + 2, p)
            if not first:
                wait_s(q)
            load_dst(k + 1, q)
            wait_idx(si[q], src[q])
            gather(q)

        pltpu.sync_copy(src_hbm.at[pl.ds(base, _MCH)], src0)
        load_src(1, 1)
        load_dst(0, 0)
        gather(0)

        step(0, 0, first=True)

        @pl.loop(0, (_MNCHUNK - 3) // 2)
        def _(t):
            k = 1 + 2 * t
            step(k, 1)
            step(k + 1, 0)

        step(_MNCHUNK - 2, 1, do_src=False)
        wait_g(0)
        wait_idx(sd[0], dst[0])
        scatter(0)
        wait_s(1)
        wait_s(0)

        plsc.subcore_barrier()
        _stripe_copy(acc_sh, msg_out.at[c], row0, s)

    mesh = plsc.VectorSubcoreMesh(core_axis_name="c", subcore_axis_name="s")
    return pl.kernel(body,
                     out_type=jax.ShapeDtypeStruct((_NC, _N, _D), jnp.float32),
                     mesh=mesh, scratch_types=scratch)


@functools.lru_cache(maxsize=None)
def _make_deg():
    scratch = [
        pltpu.VMEM((_CH,), jnp.int32),
        pltpu.VMEM((_CH, _D), jnp.float32),
        pltpu.VMEM_SHARED((_N, _D), jnp.float32),
        pltpu.SemaphoreType.DMA,
    ]

    def body(ones_hbm, dst_hbm, zeros_hbm, msg_out, dst_v, rows_v, acc_sh, sem):
        c = lax.axis_index("c")
        s = lax.axis_index("s")
        wid = s * _NC + c
        row0 = s * _STRIPE

        _stripe_copy(zeros_hbm, acc_sh, row0, s)
        pltpu.sync_copy(ones_hbm, rows_v)
        plsc.subcore_barrier()

        @pl.loop(0, _NCHUNK)
        def _(j):
            base = wid * _EPW + j * _CH
            pltpu.sync_copy(dst_hbm.at[pl.ds(base, _CH)], dst_v)
            pltpu.sync_copy(rows_v, acc_sh.at[dst_v], add=True)

        plsc.subcore_barrier()
        _stripe_copy(acc_sh, msg_out.at[c], row0, s)

    mesh = plsc.VectorSubcoreMesh(core_axis_name="c", subcore_axis_name="s")
    return pl.kernel(body,
                     out_type=jax.ShapeDtypeStruct((_NC, _N, _D), jnp.float32),
                     mesh=mesh, scratch_types=scratch)


def _tc_layer_body(h_ref, p_ref, degp_ref, wt_ref, wb_ref, b_ref, g_ref, be_ref,
                   o_ref):
    deg = degp_ref[0, :, 0:1] + degp_ref[1, :, 0:1]
    recip = 1.0 / jnp.maximum(deg, 1.0)
    hn = (p_ref[0] + p_ref[1]) * recip
    h = h_ref[...]
    z = (jnp.dot(h, wt_ref[...], preferred_element_type=jnp.float32)
         + jnp.dot(hn, wb_ref[...], preferred_element_type=jnp.float32)
         + b_ref[...])
    r = jnp.maximum(z, 0.0)
    m = jnp.mean(r, axis=0, keepdims=True)
    v = jnp.mean((r - m) * (r - m), axis=0, keepdims=True)
    o_ref[...] = h + (r - m) * lax.rsqrt(v + _EPS) * g_ref[...] + be_ref[...]


def _tc_out_body(h_ref, p_ref, degp_ref, wt_ref, wb_ref, b_ref, o_ref):
    deg = degp_ref[0, :, 0:1] + degp_ref[1, :, 0:1]
    recip = 1.0 / jnp.maximum(deg, 1.0)
    hn = (p_ref[0] + p_ref[1]) * recip
    o_ref[...] = (jnp.dot(h_ref[...], wt_ref[...],
                          preferred_element_type=jnp.float32)
                  + jnp.dot(hn, wb_ref[...], preferred_element_type=jnp.float32)
                  + b_ref[...])


_tc_layer = pl.pallas_call(
    _tc_layer_body,
    out_shape=jax.ShapeDtypeStruct((_N, _D), jnp.float32),
)

_tc_out = pl.pallas_call(
    _tc_out_body,
    out_shape=jax.ShapeDtypeStruct((_N, _NCLS), jnp.float32),
)


@jax.jit
def kernel(h, edge_index, e, W0, b0, gamma0, beta0, W1, b1, gamma1, beta1,
           W2, b2):
    del e
    src = edge_index[0]
    dst = edge_index[1]
    zeros_nd = jnp.zeros((_N, _D), jnp.float32)
    ones_ch = jnp.ones((_CH, _D), jnp.float32)

    msg = _make_msg()
    degp = _make_deg()(ones_ch, dst, zeros_nd)
    zeros_gated = zeros_nd + degp[0, :1, :1] * 0.0
    parts0 = msg(h, src, dst, zeros_gated)
    h0 = _tc_layer(h, parts0, degp, W0[:_D], W0[_D:], b0, gamma0, beta0)
    parts1 = msg(h0, src, dst, zeros_nd)
    h1 = _tc_layer(h0, parts1, degp, W1[:_D], W1[_D:], b1, gamma1, beta1)
    parts2 = msg(h1, src, dst, zeros_nd)
    return _tc_out(h1, parts2, degp, W2[:_D], W2[_D:], b2)


# speedup vs baseline: 8.2703x; 1.1418x over previous
import functools

import jax
import jax.numpy as jnp
from jax import lax
from jax.experimental import pallas as pl
from jax.experimental.pallas import tpu as pltpu
from jax.experimental.pallas import tpu_sc as plsc

_N = 10000
_E = 320000
_D = 128
_NCLS = 40
_EPS = 1e-5

_NC, _NS = 2, 16
_NW = _NC * _NS
_EPW = _E // _NW
_CH = 80
_NCHUNK = _EPW // _CH
_STRIPE = 624
_TAIL0 = _NS * _STRIPE
_TAIL = _N - _TAIL0


def _stripe_copy(src, dst, row0, s):
    pltpu.sync_copy(src.at[pl.ds(row0, _STRIPE)], dst.at[pl.ds(row0, _STRIPE)])

    @pl.when(s == 0)
    def _():
        pltpu.sync_copy(src.at[pl.ds(_TAIL0, _TAIL)],
                        dst.at[pl.ds(_TAIL0, _TAIL)])


_MCH = 40
_MNCHUNK = _EPW // _MCH
_R = 4


@functools.lru_cache(maxsize=None)
def _make_msg():
    scratch = (
        [pltpu.VMEM((_MCH,), jnp.int32) for _ in range(_R)]
        + [pltpu.VMEM((_MCH,), jnp.int32) for _ in range(_R)]
        + [pltpu.VMEM((_MCH, _D), jnp.float32) for _ in range(_R)]
        + [pltpu.VMEM_SHARED((_N, _D), jnp.float32)]
        + [pltpu.SemaphoreType.DMA] * (4 * _R)
    )

    def body(h_hbm, src_hbm, dst_hbm, zeros_hbm, msg_out, *refs):
        src = refs[0:_R]
        dst = refs[_R:2 * _R]
        rows = refs[2 * _R:3 * _R]
        acc_sh = refs[3 * _R]
        sems = refs[3 * _R + 1:]
        sg = sems[0:_R]
        ss = sems[_R:2 * _R]
        si = sems[2 * _R:3 * _R]
        sd = sems[3 * _R:4 * _R]
        c = lax.axis_index("c")
        s = lax.axis_index("s")
        wid = s * _NC + c
        row0 = s * _STRIPE
        base = wid * _EPW

        _stripe_copy(zeros_hbm, acc_sh, row0, s)
        plsc.subcore_barrier()

        def load_src(k, p):
            pltpu.async_copy(src_hbm.at[pl.ds(base + k * _MCH, _MCH)],
                             src[p], si[p])

        def load_dst(k, p):
            pltpu.async_copy(dst_hbm.at[pl.ds(base + k * _MCH, _MCH)],
                             dst[p], sd[p])

        def wait_idx(sem, buf):
            pltpu.make_async_copy(src_hbm.at[pl.ds(0, _MCH)], buf, sem).wait()

        def gather(p):
            pltpu.async_copy(h_hbm.at[src[p]], rows[p], sg[p])

        def wait_g(p):
            pltpu.make_async_copy(h_hbm.at[src[p]], rows[p], sg[p]).wait()

        def scatter(p):
            pltpu.async_copy(rows[p], acc_sh.at[dst[p]], ss[p], add=True)

        def wait_s(p):
            pltpu.make_async_copy(rows[p], acc_sh.at[dst[p]], ss[p]).wait()

        def step(k, p, first=False, do_g=True, do_src=True):
            r = (p + _R - 1) % _R
            wait_g(p)
            wait_idx(sd[p], dst[p])
            scatter(p)
            if not first:
                wait_s(r)
            if do_g:
                wait_idx(si[r], src[r])
                gather(r)
                load_dst(k + _R - 1, r)
            if do_src:
                load_src(k + _R, p)

        pltpu.sync_copy(src_hbm.at[pl.ds(base, _MCH)], src[0])
        load_src(1, 1)
        load_src(2, 2)
        load_src(3, 3)
        load_dst(0, 0)
        load_dst(1, 1)
        load_dst(2, 2)
        gather(0)
        wait_idx(si[1], src[1])
        gather(1)
        wait_idx(si[2], src[2])
        gather(2)

        step(0, 0, first=True)

        @pl.loop(0, (_MNCHUNK - 6) // _R)
        def _(t):
            k = 1 + _R * t
            for d in range(_R):
                step(k + d, (1 + d) % _R)

        step(_MNCHUNK - 5, (_MNCHUNK - 5) % _R)
        step(_MNCHUNK - 4, (_MNCHUNK - 4) % _R, do_src=False)
        step(_MNCHUNK - 3, (_MNCHUNK - 3) % _R, do_g=False, do_src=False)
        step(_MNCHUNK - 2, (_MNCHUNK - 2) % _R, do_g=False, do_src=False)
        step(_MNCHUNK - 1, (_MNCHUNK - 1) % _R, do_g=False, do_src=False)
        wait_s((_MNCHUNK - 1) % _R)

        plsc.subcore_barrier()
        _stripe_copy(acc_sh, msg_out.at[c], row0, s)

    mesh = plsc.VectorSubcoreMesh(core_axis_name="c", subcore_axis_name="s")
    return pl.kernel(body,
                     out_type=jax.ShapeDtypeStruct((_NC, _N, _D), jnp.float32),
                     mesh=mesh, scratch_types=scratch)


@functools.lru_cache(maxsize=None)
def _make_deg():
    scratch = [
        pltpu.VMEM((_CH,), jnp.int32),
        pltpu.VMEM((_CH, _D), jnp.float32),
        pltpu.VMEM_SHARED((_N, _D), jnp.float32),
        pltpu.SemaphoreType.DMA,
    ]

    def body(ones_hbm, dst_hbm, zeros_hbm, msg_out, dst_v, rows_v, acc_sh, sem):
        c = lax.axis_index("c")
        s = lax.axis_index("s")
        wid = s * _NC + c
        row0 = s * _STRIPE

        _stripe_copy(zeros_hbm, acc_sh, row0, s)
        pltpu.sync_copy(ones_hbm, rows_v)
        plsc.subcore_barrier()

        @pl.loop(0, _NCHUNK)
        def _(j):
            base = wid * _EPW + j * _CH
            pltpu.sync_copy(dst_hbm.at[pl.ds(base, _CH)], dst_v)
            pltpu.sync_copy(rows_v, acc_sh.at[dst_v], add=True)

        plsc.subcore_barrier()
        _stripe_copy(acc_sh, msg_out.at[c], row0, s)

    mesh = plsc.VectorSubcoreMesh(core_axis_name="c", subcore_axis_name="s")
    return pl.kernel(body,
                     out_type=jax.ShapeDtypeStruct((_NC, _N, _D), jnp.float32),
                     mesh=mesh, scratch_types=scratch)


def _tc_layer_body(h_ref, p_ref, degp_ref, wt_ref, wb_ref, b_ref, g_ref, be_ref,
                   o_ref):
    deg = degp_ref[0, :, 0:1] + degp_ref[1, :, 0:1]
    recip = 1.0 / jnp.maximum(deg, 1.0)
    hn = (p_ref[0] + p_ref[1]) * recip
    h = h_ref[...]
    z = (jnp.dot(h, wt_ref[...], preferred_element_type=jnp.float32)
         + jnp.dot(hn, wb_ref[...], preferred_element_type=jnp.float32)
         + b_ref[...])
    r = jnp.maximum(z, 0.0)
    m = jnp.mean(r, axis=0, keepdims=True)
    v = jnp.mean((r - m) * (r - m), axis=0, keepdims=True)
    o_ref[...] = h + (r - m) * lax.rsqrt(v + _EPS) * g_ref[...] + be_ref[...]


def _tc_out_body(h_ref, p_ref, degp_ref, wt_ref, wb_ref, b_ref, o_ref):
    deg = degp_ref[0, :, 0:1] + degp_ref[1, :, 0:1]
    recip = 1.0 / jnp.maximum(deg, 1.0)
    hn = (p_ref[0] + p_ref[1]) * recip
    o_ref[...] = (jnp.dot(h_ref[...], wt_ref[...],
                          preferred_element_type=jnp.float32)
                  + jnp.dot(hn, wb_ref[...], preferred_element_type=jnp.float32)
                  + b_ref[...])


_tc_layer = pl.pallas_call(
    _tc_layer_body,
    out_shape=jax.ShapeDtypeStruct((_N, _D), jnp.float32),
)

_tc_out = pl.pallas_call(
    _tc_out_body,
    out_shape=jax.ShapeDtypeStruct((_N, _NCLS), jnp.float32),
)


@jax.jit
def kernel(h, edge_index, e, W0, b0, gamma0, beta0, W1, b1, gamma1, beta1,
           W2, b2):
    del e
    src = edge_index[0]
    dst = edge_index[1]
    zeros_nd = jnp.zeros((_N, _D), jnp.float32)
    ones_ch = jnp.ones((_CH, _D), jnp.float32)

    msg = _make_msg()
    degp = _make_deg()(ones_ch, dst, zeros_nd)
    zeros_gated = zeros_nd + degp[0, :1, :1] * 0.0
    parts0 = msg(h, src, dst, zeros_gated)
    h0 = _tc_layer(h, parts0, degp, W0[:_D], W0[_D:], b0, gamma0, beta0)
    parts1 = msg(h0, src, dst, zeros_nd)
    h1 = _tc_layer(h0, parts1, degp, W1[:_D], W1[_D:], b1, gamma1, beta1)
    parts2 = msg(h1, src, dst, zeros_nd)
    return _tc_out(h1, parts2, degp, W2[:_D], W2[_D:], b2)
